# baseline (speedup 1.0000x reference)
import functools

import jax
import jax.numpy as jnp
from jax import lax
from jax.experimental import pallas as pl
from jax.experimental.pallas import tpu as pltpu
from jax.experimental.pallas import tpu_sc as plsc

N = 10000
E = 320000
D = 128

NC = 2
NS = 16
NW = NC * NS
L = 16

CH = 128
PER_W = 10240
NCHUNK = PER_W // CH
E_PAD = NW * PER_W

N_PAD = 10112
ROWS_PER_TILE = N_PAD // NS

N_DEG = 10240
DEG_PER_TILE = N_DEG // NS

RB = 632
GRID = N_PAD // RB

_mesh = plsc.VectorSubcoreMesh(core_axis_name="c", subcore_axis_name="s")


@functools.partial(
    pl.kernel,
    out_type=jax.ShapeDtypeStruct((NC, N_DEG), jnp.float32),
    mesh=_mesh,
    scratch_types=[
        pltpu.VMEM((NCHUNK, CH), jnp.int32),
        pltpu.VMEM((CH,), jnp.float32),
        pltpu.VMEM((DEG_PER_TILE,), jnp.float32),
        pltpu.VMEM_SHARED((N_DEG,), jnp.float32),
    ],
)
def _deg_kernel(dst_hbm, out_hbm, idx_v, ones_v, zbuf, acc):
    c = lax.axis_index("c")
    s = lax.axis_index("s")
    wid = c * NS + s

    one = jnp.ones((L,), jnp.float32)
    zero = jnp.zeros((L,), jnp.float32)
    for k in range(CH // L):
        ones_v[pl.ds(k * L, L)] = one

    def zrow(i, _):
        zbuf[pl.ds(i * L, L)] = zero
        return 0

    lax.fori_loop(0, DEG_PER_TILE // L, zrow, 0)
    pltpu.sync_copy(zbuf, acc.at[pl.ds(s * DEG_PER_TILE, DEG_PER_TILE)])
    plsc.subcore_barrier()

    pltpu.sync_copy(dst_hbm.at[wid], idx_v)

    def chunk(j, _):
        pltpu.sync_copy(ones_v, acc.at[idx_v.at[j]], add=True)
        return 0

    lax.fori_loop(0, NCHUNK, chunk, 0)
    plsc.subcore_barrier()
    pltpu.sync_copy(
        acc.at[pl.ds(s * DEG_PER_TILE, DEG_PER_TILE)],
        out_hbm.at[c].at[pl.ds(s * DEG_PER_TILE, DEG_PER_TILE)],
    )


ACH = 88
NACH = 120
PER_A = ACH * NACH
E_PAD_A = NW * PER_A
AQ = 24
NB = 3


@functools.partial(
    pl.kernel,
    out_type=jax.ShapeDtypeStruct((NC, N_PAD, D), jnp.float32),
    mesh=_mesh,
    scratch_types=[
        pltpu.VMEM((AQ, ACH), jnp.int32),
        pltpu.VMEM((AQ, ACH), jnp.int32),
        pltpu.VMEM((NB, ACH, D), jnp.float32),
        pltpu.VMEM((8, D), jnp.float32),
        pltpu.VMEM_SHARED((N_PAD, D), jnp.float32),
        [pltpu.SemaphoreType.DMA] * NB,
    ],
)
def _agg_kernel(src_hbm, dst_hbm, g_hbm, out_hbm, sidx, didx, rows, zbuf, acc, gsem):
    c = lax.axis_index("c")
    s = lax.axis_index("s")
    wid = c * NS + s

    zero = jnp.zeros((L,), jnp.float32)
    for i in range(8):
        for k in range(D // L):
            zbuf[i, pl.ds(k * L, L)] = zero

    def zrow(t, _):
        pltpu.sync_copy(zbuf, acc.at[pl.ds(s * ROWS_PER_TILE + t * 8, 8)])
        return 0

    lax.fori_loop(0, ROWS_PER_TILE // 8, zrow, 0)
    plsc.subcore_barrier()

    def section(h, _):
        pltpu.sync_copy(src_hbm.at[wid].at[pl.ds(h * AQ, AQ)], sidx)
        pltpu.sync_copy(dst_hbm.at[wid].at[pl.ds(h * AQ, AQ)], didx)

        pltpu.async_copy(g_hbm.at[sidx.at[0]], rows.at[0], gsem[0])
        pltpu.async_copy(g_hbm.at[sidx.at[1]], rows.at[1], gsem[1])

        def group(t, _):
            for b in range(NB):
                j = NB * t + b
                bw = (b + 2) % NB
                jn = jnp.minimum(j + 2, AQ - 1)
                pltpu.async_copy(g_hbm.at[sidx.at[jn]], rows.at[bw], gsem[bw])
                pltpu.make_async_copy(
                    g_hbm.at[sidx.at[j]], rows.at[b], gsem[b]
                ).wait()
                pltpu.sync_copy(rows.at[b], acc.at[didx.at[j]], add=True)
            return 0

        lax.fori_loop(0, AQ // NB, group, 0)
        pltpu.make_async_copy(g_hbm.at[sidx.at[AQ - 1]], rows.at[0], gsem[0]).wait()
        pltpu.make_async_copy(g_hbm.at[sidx.at[AQ - 1]], rows.at[1], gsem[1]).wait()
        return 0

    lax.fori_loop(0, NACH // AQ, section, 0)

    plsc.subcore_barrier()
    pltpu.sync_copy(
        acc.at[pl.ds(s * ROWS_PER_TILE, ROWS_PER_TILE)],
        out_hbm.at[c].at[pl.ds(s * ROWS_PER_TILE, ROWS_PER_TILE)],
    )


def _dis(degp):
    return lax.rsqrt(degp[:, 0] + degp[:, 1] + 1.0)


def _scale1_body(x_ref, w_ref, deg_ref, o_ref):
    dis = _dis(deg_ref[...])
    h = jnp.dot(x_ref[...], w_ref[...], preferred_element_type=jnp.float32)
    o_ref[...] = h * dis[:, None]


def _layer2_body(u_ref, g_ref, deg_ref, w_ref, b_ref, o_ref):
    dis = _dis(deg_ref[...])
    u = u_ref[0] + u_ref[1] + g_ref[...]
    h = jnp.maximum(u * dis[:, None] + b_ref[...], 0.0)
    o_ref[...] = jnp.dot(h, w_ref[...], preferred_element_type=jnp.float32) * dis[:, None]


def _final_body(u_ref, g_ref, deg_ref, b_ref, o_ref):
    dis = _dis(deg_ref[...])
    u = u_ref[0] + u_ref[1] + g_ref[...]
    o_ref[...] = u * dis[:, None] + b_ref[...]


_row_spec = pl.BlockSpec((RB, D), lambda i: (i, 0))
_deg_spec = pl.BlockSpec((RB, NC), lambda i: (i, 0))
_u_spec = pl.BlockSpec((NC, RB, D), lambda i: (0, i, 0))
_w_spec = pl.BlockSpec((D, D), lambda i: (0, 0))
_b_spec = pl.BlockSpec((1, D), lambda i: (0, 0))
_out_shape = jax.ShapeDtypeStruct((N_PAD, D), jnp.float32)

_scale1 = pl.pallas_call(
    _scale1_body,
    grid=(GRID,),
    in_specs=[_row_spec, _w_spec, _deg_spec],
    out_specs=_row_spec,
    out_shape=_out_shape,
)

_layer2 = pl.pallas_call(
    _layer2_body,
    grid=(GRID,),
    in_specs=[_u_spec, _row_spec, _deg_spec, _w_spec, _b_spec],
    out_specs=_row_spec,
    out_shape=_out_shape,
)

_final = pl.pallas_call(
    _final_body,
    grid=(GRID,),
    in_specs=[_u_spec, _row_spec, _deg_spec, _b_spec],
    out_specs=_row_spec,
    out_shape=_out_shape,
)


@jax.jit
def kernel(x, edge_index, W1, b1, W2, b2):
    ei = edge_index.astype(jnp.int32)
    pad = jnp.full((E_PAD - E,), N, jnp.int32)
    pad_a = jnp.full((E_PAD_A - E,), N, jnp.int32)
    src_a = jnp.concatenate([ei[0], pad_a]).reshape(NW, NACH, ACH)
    dst_a = jnp.concatenate([ei[1], pad_a]).reshape(NW, NACH, ACH)
    dst = jnp.concatenate([ei[1], pad]).reshape(NW, NCHUNK, CH)
    x_pad = jnp.zeros((N_PAD, D), jnp.float32).at[:N].set(x)
    b1r = b1.reshape(1, D)
    b2r = b2.reshape(1, D)

    degp = _deg_kernel(dst).T
    g1 = _scale1(x_pad, W1, degp)
    u1 = _agg_kernel(src_a, dst_a, g1)
    g2 = _layer2(u1, g1, degp, W2, b1r)
    u2 = _agg_kernel(src_a, dst_a, g2)
    out = _final(u2, g2, degp, b2r)
    return out[:N]


# speedup vs baseline: 1.9951x; 1.9951x over previous
import functools

import jax
import jax.numpy as jnp
from jax import lax
from jax.experimental import pallas as pl
from jax.experimental.pallas import tpu as pltpu
from jax.experimental.pallas import tpu_sc as plsc

N = 10000
E = 320000
D = 128

NC = 2
NS = 16
NW = NC * NS
L = 16

CH = 128
PER_W = 10240
NCHUNK = PER_W // CH
E_PAD = NW * PER_W

N_PAD = 10112
ROWS_PER_TILE = N_PAD // NS

N_DEG = 10240
DEG_PER_TILE = N_DEG // NS

RB = 632
GRID = N_PAD // RB

_mesh = plsc.VectorSubcoreMesh(core_axis_name="c", subcore_axis_name="s")


@functools.partial(
    pl.kernel,
    out_type=jax.ShapeDtypeStruct((NC, N_DEG), jnp.float32),
    mesh=_mesh,
    scratch_types=[
        pltpu.VMEM((NCHUNK, CH), jnp.int32),
        pltpu.VMEM((CH,), jnp.float32),
        pltpu.VMEM((DEG_PER_TILE,), jnp.float32),
        pltpu.VMEM_SHARED((N_DEG,), jnp.float32),
    ],
)
def _deg_kernel(dst_hbm, out_hbm, idx_v, ones_v, zbuf, acc):
    c = lax.axis_index("c")
    s = lax.axis_index("s")
    wid = c * NS + s

    one = jnp.ones((L,), jnp.float32)
    zero = jnp.zeros((L,), jnp.float32)
    for k in range(CH // L):
        ones_v[pl.ds(k * L, L)] = one

    def zrow(i, _):
        zbuf[pl.ds(i * L, L)] = zero
        return 0

    lax.fori_loop(0, DEG_PER_TILE // L, zrow, 0)
    pltpu.sync_copy(zbuf, acc.at[pl.ds(s * DEG_PER_TILE, DEG_PER_TILE)])
    plsc.subcore_barrier()

    pltpu.sync_copy(dst_hbm.at[wid], idx_v)

    def chunk(j, _):
        pltpu.sync_copy(ones_v, acc.at[idx_v.at[j]], add=True)
        return 0

    lax.fori_loop(0, NCHUNK, chunk, 0)
    plsc.subcore_barrier()
    pltpu.sync_copy(
        acc.at[pl.ds(s * DEG_PER_TILE, DEG_PER_TILE)],
        out_hbm.at[c].at[pl.ds(s * DEG_PER_TILE, DEG_PER_TILE)],
    )


NH = NCHUNK // 2


@functools.partial(
    pl.kernel,
    out_type=jax.ShapeDtypeStruct((NC, N_PAD, D), jnp.float32),
    mesh=_mesh,
    scratch_types=[
        pltpu.VMEM((NH, CH), jnp.int32),
        pltpu.VMEM((NH, CH), jnp.int32),
        pltpu.VMEM((2, CH, D), jnp.float32),
        pltpu.VMEM((8, D), jnp.float32),
        pltpu.VMEM_SHARED((N_PAD, D), jnp.float32),
        pltpu.SemaphoreType.DMA,
        pltpu.SemaphoreType.DMA,
    ],
)
def _agg_kernel(src_hbm, dst_hbm, g_hbm, out_hbm, sidx, didx, rows, zbuf, acc, sem0, sem1):
    c = lax.axis_index("c")
    s = lax.axis_index("s")
    wid = c * NS + s

    zero = jnp.zeros((L,), jnp.float32)
    for i in range(8):
        for k in range(D // L):
            zbuf[i, pl.ds(k * L, L)] = zero

    def zrow(t, _):
        pltpu.sync_copy(zbuf, acc.at[pl.ds(s * ROWS_PER_TILE + t * 8, 8)])
        return 0

    lax.fori_loop(0, ROWS_PER_TILE // 8, zrow, 0)
    plsc.subcore_barrier()

    for h in range(2):
        pltpu.sync_copy(src_hbm.at[wid].at[pl.ds(h * NH, NH)], sidx)
        pltpu.sync_copy(dst_hbm.at[wid].at[pl.ds(h * NH, NH)], didx)

        pltpu.async_copy(g_hbm.at[sidx.at[0]], rows.at[0], sem0)

        def pair(t, _):
            j0 = 2 * t
            pltpu.async_copy(g_hbm.at[sidx.at[j0 + 1]], rows.at[1], sem1)
            pltpu.make_async_copy(g_hbm.at[sidx.at[j0]], rows.at[0], sem0).wait()
            pltpu.sync_copy(rows.at[0], acc.at[didx.at[j0]], add=True)
            jn = jnp.minimum(j0 + 2, NH - 1)
            pltpu.async_copy(g_hbm.at[sidx.at[jn]], rows.at[0], sem0)
            pltpu.make_async_copy(g_hbm.at[sidx.at[j0 + 1]], rows.at[1], sem1).wait()
            pltpu.sync_copy(rows.at[1], acc.at[didx.at[j0 + 1]], add=True)
            return 0

        lax.fori_loop(0, NH // 2, pair, 0)
        pltpu.make_async_copy(g_hbm.at[sidx.at[NH - 1]], rows.at[0], sem0).wait()

    plsc.subcore_barrier()
    pltpu.sync_copy(
        acc.at[pl.ds(s * ROWS_PER_TILE, ROWS_PER_TILE)],
        out_hbm.at[c].at[pl.ds(s * ROWS_PER_TILE, ROWS_PER_TILE)],
    )


def _dis(degp):
    return lax.rsqrt(degp[:, 0] + degp[:, 1] + 1.0)


def _scale1_body(x_ref, w_ref, deg_ref, o_ref):
    dis = _dis(deg_ref[...])
    h = jnp.dot(x_ref[...], w_ref[...], preferred_element_type=jnp.float32)
    o_ref[...] = h * dis[:, None]


def _layer2_body(u_ref, g_ref, deg_ref, w_ref, b_ref, o_ref):
    dis = _dis(deg_ref[...])
    u = u_ref[0] + u_ref[1] + g_ref[...]
    h = jnp.maximum(u * dis[:, None] + b_ref[...], 0.0)
    o_ref[...] = jnp.dot(h, w_ref[...], preferred_element_type=jnp.float32) * dis[:, None]


def _final_body(u_ref, g_ref, deg_ref, b_ref, o_ref):
    dis = _dis(deg_ref[...])
    u = u_ref[0] + u_ref[1] + g_ref[...]
    o_ref[...] = u * dis[:, None] + b_ref[...]


_row_spec = pl.BlockSpec((RB, D), lambda i: (i, 0))
_deg_spec = pl.BlockSpec((RB, NC), lambda i: (i, 0))
_u_spec = pl.BlockSpec((NC, RB, D), lambda i: (0, i, 0))
_w_spec = pl.BlockSpec((D, D), lambda i: (0, 0))
_b_spec = pl.BlockSpec((1, D), lambda i: (0, 0))
_out_shape = jax.ShapeDtypeStruct((N_PAD, D), jnp.float32)

_scale1 = pl.pallas_call(
    _scale1_body,
    grid=(GRID,),
    in_specs=[_row_spec, _w_spec, _deg_spec],
    out_specs=_row_spec,
    out_shape=_out_shape,
)

_layer2 = pl.pallas_call(
    _layer2_body,
    grid=(GRID,),
    in_specs=[_u_spec, _row_spec, _deg_spec, _w_spec, _b_spec],
    out_specs=_row_spec,
    out_shape=_out_shape,
)

_final = pl.pallas_call(
    _final_body,
    grid=(GRID,),
    in_specs=[_u_spec, _row_spec, _deg_spec, _b_spec],
    out_specs=_row_spec,
    out_shape=_out_shape,
)


@jax.jit
def kernel(x, edge_index, W1, b1, W2, b2):
    ei = edge_index.astype(jnp.int32)
    pad = jnp.full((E_PAD - E,), N, jnp.int32)
    src = jnp.concatenate([ei[0], pad]).reshape(NW, NCHUNK, CH)
    dst = jnp.concatenate([ei[1], pad]).reshape(NW, NCHUNK, CH)
    x_pad = jnp.zeros((N_PAD, D), jnp.float32).at[:N].set(x)
    b1r = b1.reshape(1, D)
    b2r = b2.reshape(1, D)

    degp = _deg_kernel(dst).T
    g1 = _scale1(x_pad, W1, degp)
    u1 = _agg_kernel(src, dst, g1)
    g2 = _layer2(u1, g1, degp, W2, b1r)
    u2 = _agg_kernel(src, dst, g2)
    out = _final(u2, g2, degp, b2r)
    return out[:N]
